# baseline (speedup 1.0000x reference)
import functools

import jax
import jax.numpy as jnp
from jax import lax
from jax.experimental import pallas as pl
from jax.experimental.pallas import tpu as pltpu
from jax.experimental.pallas import tpu_sc as plsc

E = 320000
D = 128
N = 10000

NC = 2
NS = 16
NW = NC * NS
L = 16

CH = 80
ROWS2 = E // CH
ROWS2P = 4096
RPW = ROWS2P // NW
RPS = ROWS2P // NS
NPAD = 10240
RPT = NPAD // NS

HALF = 32

BM = 2560
GRID_E = E // BM


def _make_score_body(bm_rows):
  def _score_body(m_ref, v_ref, w0_ref, w1_ref, b_ref, aw_ref, ab_ref,
                  s_ref, g_ref):
    u = jnp.dot(m_ref[...], w0_ref[...], preferred_element_type=jnp.float32)
    u1 = jnp.dot(v_ref[...], w1_ref[...], preferred_element_type=jnp.float32)
    z = u + u1 + b_ref[...]
    z = jnp.where(z >= 0, z, 0.2 * z)
    s = jnp.sum(z * aw_ref[...], axis=1) + ab_ref[0, 0]
    s_ref[0] = s.reshape(bm_rows // 128, 128)
    bmx = jnp.full((8, 128), jnp.max(s))
    i = pl.program_id(0)

    @pl.when(i == 0)
    def _():
      g_ref[...] = bmx

    @pl.when(i != 0)
    def _():
      g_ref[...] = jnp.maximum(g_ref[...], bmx)

  return _score_body


def _score_half(m, v, w0t, w1t, b01, aw, ab, bm_rows, grid_n, blk_off):
  return pl.pallas_call(
      _make_score_body(bm_rows),
      grid=(grid_n,),
      in_specs=[
          pl.BlockSpec((bm_rows, D), lambda i: (i + blk_off, 0)),
          pl.BlockSpec((bm_rows, D), lambda i: (i, 0)),
          pl.BlockSpec((D, D), lambda i: (0, 0)),
          pl.BlockSpec((D, D), lambda i: (0, 0)),
          pl.BlockSpec((1, D), lambda i: (0, 0)),
          pl.BlockSpec((1, D), lambda i: (0, 0)),
          pl.BlockSpec(memory_space=pltpu.SMEM),
      ],
      out_specs=[
          pl.BlockSpec((1, bm_rows // 128, 128), lambda i: (i, 0, 0)),
          pl.BlockSpec((8, 128), lambda i: (0, 0)),
      ],
      out_shape=[
          jax.ShapeDtypeStruct((grid_n, bm_rows // 128, 128), jnp.float32),
          jax.ShapeDtypeStruct((8, 128), jnp.float32),
      ],
  )(m, v, w0t, w1t, b01, aw, ab)


def _combine_body(p_ref, o_ref):
  o_ref[...] = p_ref[0] + p_ref[1]


def _make_gather_body(row_lo, nrows):
  rpw = nrows // NW

  def _gather_body(p_hbm, rev_hbm, v_hbm, idx_v, rows4, gsem, wsem):
    c = lax.axis_index("c")
    s = lax.axis_index("s")
    w = s * NC + c
    base_row = row_lo + w * rpw
    nv = jnp.clip(ROWS2 - base_row, 0, rpw)
    pltpu.sync_copy(rev_hbm.at[pl.ds(base_row, rpw)], idx_v)

    for b in range(3):
      @pl.when(b < nv)
      def _():
        pltpu.async_copy(p_hbm.at[idx_v.at[b]], rows4.at[b], gsem)

    @pl.loop(0, nv)
    def _(j):
      bj = lax.rem(j, 4)
      pltpu.make_async_copy(p_hbm.at[pl.ds(0, CH)], rows4.at[bj], gsem).wait()
      pltpu.async_copy(rows4.at[bj],
                       v_hbm.at[pl.ds((base_row - row_lo + j) * CH, CH)], wsem)

      @pl.when(j >= 1)
      def _():
        pltpu.make_async_copy(
            rows4.at[bj], v_hbm.at[pl.ds(0, CH)], wsem).wait()

      @pl.when(j + 3 < nv)
      def _():
        b2 = lax.rem(j + 3, 4)
        pltpu.async_copy(p_hbm.at[idx_v.at[j + 3]], rows4.at[b2], gsem)

    @pl.when(nv > 0)
    def _():
      pltpu.make_async_copy(
          rows4.at[0], v_hbm.at[pl.ds(0, CH)], wsem).wait()

  return _gather_body


def _soft_body(s_hbm, dest_hbm, g_hbm, alpha_hbm, seg_sm, dv, ev, av, zseg,
               den0, den1, gbuf, ssem, dsem):
  c = lax.axis_index("c")
  s = lax.axis_index("s")
  w = s * NC + c

  pltpu.sync_copy(g_hbm, gbuf)
  g16 = gbuf[...]
  zeros16 = jnp.zeros((L,), jnp.float32)
  for k in range(RPT // L):
    zseg[pl.ds(k * L, L)] = zeros16
  pltpu.sync_copy(zseg, seg_sm.at[pl.ds(s * RPT, RPT)])
  plsc.subcore_barrier()

  pltpu.sync_copy(s_hbm.at[pl.ds(s * RPS, RPS)], ev)
  pltpu.sync_copy(dest_hbm.at[pl.ds(s * RPS, RPS)], dv)

  @pl.loop(0, RPS)
  def _(k):
    for t in range(CH // L):
      ev[k, pl.ds(t * L, L)] = jnp.exp(ev[k, pl.ds(t * L, L)] - g16)
    pltpu.async_copy(ev.at[k], seg_sm.at[dv.at[k]], ssem, add=True)

    @pl.when(k >= 8)
    def _():
      pltpu.make_async_copy(ev.at[0], seg_sm.at[pl.ds(0, CH)], ssem).wait()

  for _ in range(8):
    pltpu.make_async_copy(ev.at[0], seg_sm.at[pl.ds(0, CH)], ssem).wait()

  plsc.subcore_barrier()

  base_row = w * RPW
  pltpu.sync_copy(s_hbm.at[pl.ds(base_row, RPW)], ev.at[pl.ds(0, RPW)])
  pltpu.sync_copy(dest_hbm.at[pl.ds(base_row, RPW)], dv.at[pl.ds(0, RPW)])
  dens = (den0, den1)
  pltpu.async_copy(seg_sm.at[dv.at[0]], den0, dsem)

  @pl.loop(0, RPW // 2)
  def _(jj):
    for k2 in range(2):
      j = 2 * jj + k2
      den = dens[k2]
      nxt = dens[1 - k2]
      pltpu.make_async_copy(seg_sm.at[pl.ds(0, CH)], den, dsem).wait()

      @pl.when(j + 1 < RPW)
      def _():
        pltpu.async_copy(seg_sm.at[dv.at[j + 1]], nxt, dsem)

      for t in range(CH // L):
        d16 = den[pl.ds(t * L, L)]
        e16 = jnp.exp(ev[j, pl.ds(t * L, L)] - g16)
        av[j, pl.ds(t * L, L)] = e16 / d16

  pltpu.sync_copy(av, alpha_hbm.at[pl.ds(base_row, RPW)])


def _scat_body(alpha_hbm, dest_hbm, m_hbm, part_hbm, out_sm, dv, av,
               mb0, mb1, mb2, lsem, ssem):
  c = lax.axis_index("c")
  s = lax.axis_index("s")
  w = s * NC + c
  mbufs = (mb0, mb1, mb2)

  zeros16 = jnp.zeros((L,), jnp.float32)

  @pl.loop(0, CH)
  def _(r):
    for q in range(D // L):
      mb0[r, pl.ds(q * L, L)] = zeros16

  for k in range(RPT // CH):
    pltpu.sync_copy(mb0, out_sm.at[pl.ds(s * RPT + k * CH, CH)])
  plsc.subcore_barrier()

  base_row = w * RPW
  nv = jnp.minimum(RPW, ROWS2 - base_row)

  for h in range(RPW // HALF):
    hbase = base_row + h * HALF
    nvh = jnp.clip(nv - h * HALF, 0, HALF)

    @pl.when(nvh > 0)
    def _():
      pltpu.sync_copy(alpha_hbm.at[pl.ds(hbase, HALF)], av)
      pltpu.sync_copy(dest_hbm.at[pl.ds(hbase, HALF)], dv)
      pltpu.async_copy(m_hbm.at[pl.ds(hbase * CH, CH)], mb0, lsem)

      @pl.loop(0, (HALF + 2) // 3)
      def _(jj):
        for k2 in range(3):
          j = 3 * jj + k2
          mb = mbufs[k2]

          @pl.when(j < HALF)
          def _():
            pltpu.make_async_copy(m_hbm.at[pl.ds(0, CH)], mb, lsem).wait()

            @pl.when(j >= 2)
            def _():
              pltpu.make_async_copy(
                  mb, out_sm.at[pl.ds(0, CH)], ssem).wait()

            @pl.when(j + 1 < HALF)
            def _():
              nxt = mbufs[(k2 + 1) % 3]
              pltpu.async_copy(m_hbm.at[pl.ds((hbase + j + 1) * CH, CH)],
                               nxt, lsem)

            def _scale_rows(g, carry):
              a16 = av[j, pl.ds(g * L, L)]
              for k in range(L):
                a_s = jnp.full((L,), a16[k], jnp.float32)
                r = g * L + k
                for q in range(D // L):
                  mb[r, pl.ds(q * L, L)] = mb[r, pl.ds(q * L, L)] * a_s
              return carry

            lax.fori_loop(0, CH // L, _scale_rows, 0)
            pltpu.async_copy(mb, out_sm.at[dv.at[j]], ssem, add=True)

      for _ in range(2):
        pltpu.make_async_copy(mb0, out_sm.at[pl.ds(0, CH)], ssem).wait()

  plsc.subcore_barrier()

  pltpu.sync_copy(out_sm.at[pl.ds(s * RPT, RPT)],
                  part_hbm.at[c].at[pl.ds(s * RPT, RPT)])


_MESH = plsc.VectorSubcoreMesh(core_axis_name="c", subcore_axis_name="s",
                               num_cores=NC, num_subcores=NS)

ROWS_H = ROWS2P // 2
EA = ROWS_H * CH
EB = E - EA
BMA, GRID_A = 2048, EA // 2048
BMB, GRID_B = 1280, EB // 1280

_gather_call_a = pl.kernel(
    _make_gather_body(0, ROWS_H),
    out_type=jax.ShapeDtypeStruct((EA, D), jnp.float32),
    mesh=_MESH,
    scratch_types=[
        pltpu.VMEM((ROWS_H // NW, CH), jnp.int32),
        pltpu.VMEM((4, CH, D), jnp.float32),
        pltpu.SemaphoreType.DMA,
        pltpu.SemaphoreType.DMA,
    ],
)

_gather_call_b = pl.kernel(
    _make_gather_body(ROWS_H, ROWS_H),
    out_type=jax.ShapeDtypeStruct((EB, D), jnp.float32),
    mesh=_MESH,
    scratch_types=[
        pltpu.VMEM((ROWS_H // NW, CH), jnp.int32),
        pltpu.VMEM((4, CH, D), jnp.float32),
        pltpu.SemaphoreType.DMA,
        pltpu.SemaphoreType.DMA,
    ],
)

_soft_call = pl.kernel(
    _soft_body,
    out_type=jax.ShapeDtypeStruct((ROWS2P, CH), jnp.float32),
    mesh=_MESH,
    scratch_types=[
        pltpu.VMEM_SHARED((NPAD,), jnp.float32),
        pltpu.VMEM((RPS, CH), jnp.int32),
        pltpu.VMEM((RPS, CH), jnp.float32),
        pltpu.VMEM((RPW, CH), jnp.float32),
        pltpu.VMEM((RPT,), jnp.float32),
        pltpu.VMEM((CH,), jnp.float32),
        pltpu.VMEM((CH,), jnp.float32),
        pltpu.VMEM((L,), jnp.float32),
        pltpu.SemaphoreType.DMA,
        pltpu.SemaphoreType.DMA,
    ],
)

_scat_call = pl.kernel(
    _scat_body,
    out_type=jax.ShapeDtypeStruct((NC, NPAD, D), jnp.float32),
    mesh=_MESH,
    scratch_types=[
        pltpu.VMEM_SHARED((NPAD, D), jnp.float32),
        pltpu.VMEM((HALF, CH), jnp.int32),
        pltpu.VMEM((HALF, CH), jnp.float32),
        pltpu.VMEM((CH, D), jnp.float32),
        pltpu.VMEM((CH, D), jnp.float32),
        pltpu.VMEM((CH, D), jnp.float32),
        pltpu.SemaphoreType.DMA,
        pltpu.SemaphoreType.DMA,
    ],
)


@jax.jit
def _run(M, dest32, rev32, W0_w, W0_b, W1_w, W1_b, a_w, a_b):
  w1t = W1_w.T
  w0t = W0_w.T
  b01 = (W0_b + W1_b).reshape(1, D)
  aw = a_w.reshape(1, D)
  ab = a_b.reshape(1, 1)

  pad = ROWS2P * CH - E
  rev_p = jnp.concatenate(
      [rev32, jnp.zeros((pad,), jnp.int32)]).reshape(ROWS2P, CH)
  v_a = _gather_call_a(M, rev_p)
  v_b = _gather_call_b(M, rev_p)

  s_a, gmax_a = _score_half(M, v_a, w0t, w1t, b01, aw, ab, BMA, GRID_A, 0)
  s_b, gmax_b = _score_half(M, v_b, w0t, w1t, b01, aw, ab, BMB, GRID_B,
                            EA // BMB)
  gmax = jnp.maximum(gmax_a, gmax_b)

  s_p = jnp.concatenate(
      [s_a.reshape(EA), s_b.reshape(EB),
       jnp.full((pad,), -1e30, jnp.float32)]
  ).reshape(ROWS2P, CH)
  dest_p = jnp.concatenate(
      [dest32, jnp.zeros((pad,), jnp.int32)]).reshape(ROWS2P, CH)
  g16 = gmax.reshape(-1)[:L]
  alpha2 = _soft_call(s_p, dest_p, g16)
  partials = _scat_call(alpha2, dest_p, M)

  out = pl.pallas_call(
      _combine_body,
      grid=(5,),
      in_specs=[pl.BlockSpec((NC, 2000, D), lambda i: (0, i, 0))],
      out_specs=pl.BlockSpec((2000, D), lambda i: (i, 0)),
      out_shape=jax.ShapeDtypeStruct((N, D), jnp.float32),
  )(partials)

  return out, alpha2.reshape(-1)[:E].reshape(E, 1)


def kernel(M, dest, dim_size, rev_index, W0_w, W0_b, W1_w, W1_b, a_w, a_b):
  dest32 = dest.astype(jnp.int32)
  rev32 = rev_index.astype(jnp.int32)
  return _run(M, dest32, rev32, W0_w, W0_b, W1_w, W1_b, a_w, a_b)


# speedup vs baseline: 1.0579x; 1.0579x over previous
import functools

import jax
import jax.numpy as jnp
from jax import lax
from jax.experimental import pallas as pl
from jax.experimental.pallas import tpu as pltpu
from jax.experimental.pallas import tpu_sc as plsc

E = 320000
D = 128
N = 10000

NC = 2
NS = 16
NW = NC * NS
L = 16

CH = 80
ROWS2 = E // CH
ROWS2P = 4096
RPW = ROWS2P // NW
RPS = ROWS2P // NS
NPAD = 10240
RPT = NPAD // NS

HALF = 32

BM = 2560
GRID_E = E // BM


def _make_score_body(bm_rows):
  def _score_body(m_ref, v_ref, w0_ref, w1_ref, b_ref, aw_ref, ab_ref,
                  s_ref, g_ref):
    u = jnp.dot(m_ref[...], w0_ref[...], preferred_element_type=jnp.float32)
    u1 = jnp.dot(v_ref[...], w1_ref[...], preferred_element_type=jnp.float32)
    z = u + u1 + b_ref[...]
    z = jnp.where(z >= 0, z, 0.2 * z)
    s = jnp.sum(z * aw_ref[...], axis=1) + ab_ref[0, 0]
    s_ref[0] = s.reshape(bm_rows // 128, 128)
    bmx = jnp.full((8, 128), jnp.max(s))
    i = pl.program_id(0)

    @pl.when(i == 0)
    def _():
      g_ref[...] = bmx

    @pl.when(i != 0)
    def _():
      g_ref[...] = jnp.maximum(g_ref[...], bmx)

  return _score_body


def _score_half(m, v, w0t, w1t, b01, aw, ab, bm_rows, grid_n, blk_off):
  return pl.pallas_call(
      _make_score_body(bm_rows),
      grid=(grid_n,),
      in_specs=[
          pl.BlockSpec((bm_rows, D), lambda i: (i + blk_off, 0)),
          pl.BlockSpec((bm_rows, D), lambda i: (i, 0)),
          pl.BlockSpec((D, D), lambda i: (0, 0)),
          pl.BlockSpec((D, D), lambda i: (0, 0)),
          pl.BlockSpec((1, D), lambda i: (0, 0)),
          pl.BlockSpec((1, D), lambda i: (0, 0)),
          pl.BlockSpec(memory_space=pltpu.SMEM),
      ],
      out_specs=[
          pl.BlockSpec((1, bm_rows // 128, 128), lambda i: (i, 0, 0)),
          pl.BlockSpec((8, 128), lambda i: (0, 0)),
      ],
      out_shape=[
          jax.ShapeDtypeStruct((grid_n, bm_rows // 128, 128), jnp.float32),
          jax.ShapeDtypeStruct((8, 128), jnp.float32),
      ],
  )(m, v, w0t, w1t, b01, aw, ab)


def _combine_body(p_ref, o_ref):
  o_ref[...] = p_ref[0] + p_ref[1]


def _gather_body(p_hbm, rev_hbm, v_hbm, idx_v, rows4, gsem, wsem):
  c = lax.axis_index("c")
  s = lax.axis_index("s")
  w = s * NC + c
  base_row = w * RPW
  nv = jnp.minimum(RPW, ROWS2 - base_row)
  pltpu.sync_copy(rev_hbm.at[pl.ds(base_row, RPW)], idx_v)

  for b in range(3):
    pltpu.async_copy(p_hbm.at[idx_v.at[b]], rows4.at[b], gsem)

  @pl.loop(0, nv)
  def _(j):
    bj = lax.rem(j, 4)
    pltpu.make_async_copy(p_hbm.at[pl.ds(0, CH)], rows4.at[bj], gsem).wait()
    pltpu.async_copy(rows4.at[bj],
                     v_hbm.at[pl.ds((base_row + j) * CH, CH)], wsem)

    @pl.when(j >= 1)
    def _():
      pltpu.make_async_copy(
          rows4.at[bj], v_hbm.at[pl.ds(0, CH)], wsem).wait()

    @pl.when(j + 3 < nv)
    def _():
      b2 = lax.rem(j + 3, 4)
      pltpu.async_copy(p_hbm.at[idx_v.at[j + 3]], rows4.at[b2], gsem)

  pltpu.make_async_copy(rows4.at[0], v_hbm.at[pl.ds(0, CH)], wsem).wait()


def _soft_body(s_hbm, dest_hbm, g_hbm, alpha_hbm, seg_sm, dv, ev, av, zseg,
               den0, den1, gbuf, ssem, dsem):
  c = lax.axis_index("c")
  s = lax.axis_index("s")
  w = s * NC + c

  pltpu.sync_copy(g_hbm, gbuf)
  g16 = gbuf[...]
  zeros16 = jnp.zeros((L,), jnp.float32)
  for k in range(RPT // L):
    zseg[pl.ds(k * L, L)] = zeros16
  pltpu.sync_copy(zseg, seg_sm.at[pl.ds(s * RPT, RPT)])
  plsc.subcore_barrier()

  pltpu.sync_copy(s_hbm.at[pl.ds(s * RPS, RPS)], ev)
  pltpu.sync_copy(dest_hbm.at[pl.ds(s * RPS, RPS)], dv)

  @pl.loop(0, RPS)
  def _(k):
    for t in range(CH // L):
      ev[k, pl.ds(t * L, L)] = jnp.exp(ev[k, pl.ds(t * L, L)] - g16)
    pltpu.async_copy(ev.at[k], seg_sm.at[dv.at[k]], ssem, add=True)

    @pl.when(k >= 8)
    def _():
      pltpu.make_async_copy(ev.at[0], seg_sm.at[pl.ds(0, CH)], ssem).wait()

  for _ in range(8):
    pltpu.make_async_copy(ev.at[0], seg_sm.at[pl.ds(0, CH)], ssem).wait()

  plsc.subcore_barrier()

  base_row = w * RPW
  pltpu.sync_copy(s_hbm.at[pl.ds(base_row, RPW)], ev.at[pl.ds(0, RPW)])
  pltpu.sync_copy(dest_hbm.at[pl.ds(base_row, RPW)], dv.at[pl.ds(0, RPW)])
  dens = (den0, den1)
  pltpu.async_copy(seg_sm.at[dv.at[0]], den0, dsem)

  @pl.loop(0, RPW // 2)
  def _(jj):
    for k2 in range(2):
      j = 2 * jj + k2
      den = dens[k2]
      nxt = dens[1 - k2]
      pltpu.make_async_copy(seg_sm.at[pl.ds(0, CH)], den, dsem).wait()

      @pl.when(j + 1 < RPW)
      def _():
        pltpu.async_copy(seg_sm.at[dv.at[j + 1]], nxt, dsem)

      for t in range(CH // L):
        d16 = den[pl.ds(t * L, L)]
        e16 = jnp.exp(ev[j, pl.ds(t * L, L)] - g16)
        av[j, pl.ds(t * L, L)] = e16 / d16

  pltpu.sync_copy(av, alpha_hbm.at[pl.ds(base_row, RPW)])


def _scat_body(alpha_hbm, dest_hbm, m_hbm, part_hbm, out_sm, dv, av,
               mb0, mb1, mb2, lsem, ssem):
  c = lax.axis_index("c")
  s = lax.axis_index("s")
  w = s * NC + c
  mbufs = (mb0, mb1, mb2)

  zeros16 = jnp.zeros((L,), jnp.float32)

  @pl.loop(0, CH)
  def _(r):
    for q in range(D // L):
      mb0[r, pl.ds(q * L, L)] = zeros16

  for k in range(RPT // CH):
    pltpu.sync_copy(mb0, out_sm.at[pl.ds(s * RPT + k * CH, CH)])
  plsc.subcore_barrier()

  base_row = w * RPW
  nv = jnp.minimum(RPW, ROWS2 - base_row)

  for h in range(RPW // HALF):
    hbase = base_row + h * HALF
    nvh = jnp.clip(nv - h * HALF, 0, HALF)

    @pl.when(nvh > 0)
    def _():
      pltpu.sync_copy(alpha_hbm.at[pl.ds(hbase, HALF)], av)
      pltpu.sync_copy(dest_hbm.at[pl.ds(hbase, HALF)], dv)
      pltpu.async_copy(m_hbm.at[pl.ds(hbase * CH, CH)], mb0, lsem)

      @pl.loop(0, (HALF + 2) // 3)
      def _(jj):
        for k2 in range(3):
          j = 3 * jj + k2
          mb = mbufs[k2]

          @pl.when(j < HALF)
          def _():
            pltpu.make_async_copy(m_hbm.at[pl.ds(0, CH)], mb, lsem).wait()

            @pl.when(j >= 2)
            def _():
              pltpu.make_async_copy(
                  mb, out_sm.at[pl.ds(0, CH)], ssem).wait()

            @pl.when(j + 1 < HALF)
            def _():
              nxt = mbufs[(k2 + 1) % 3]
              pltpu.async_copy(m_hbm.at[pl.ds((hbase + j + 1) * CH, CH)],
                               nxt, lsem)

            def _scale_rows(g, carry):
              a16 = av[j, pl.ds(g * L, L)]
              for k in range(L):
                a_s = jnp.full((L,), a16[k], jnp.float32)
                r = g * L + k
                for q in range(D // L):
                  mb[r, pl.ds(q * L, L)] = mb[r, pl.ds(q * L, L)] * a_s
              return carry

            lax.fori_loop(0, CH // L, _scale_rows, 0)
            pltpu.async_copy(mb, out_sm.at[dv.at[j]], ssem, add=True)

      for _ in range(2):
        pltpu.make_async_copy(mb0, out_sm.at[pl.ds(0, CH)], ssem).wait()

  plsc.subcore_barrier()

  pltpu.sync_copy(out_sm.at[pl.ds(s * RPT, RPT)],
                  part_hbm.at[c].at[pl.ds(s * RPT, RPT)])


_MESH = plsc.VectorSubcoreMesh(core_axis_name="c", subcore_axis_name="s",
                               num_cores=NC, num_subcores=NS)

_gather_call = pl.kernel(
    _gather_body,
    out_type=jax.ShapeDtypeStruct((E, D), jnp.float32),
    mesh=_MESH,
    scratch_types=[
        pltpu.VMEM((RPW, CH), jnp.int32),
        pltpu.VMEM((4, CH, D), jnp.float32),
        pltpu.SemaphoreType.DMA,
        pltpu.SemaphoreType.DMA,
    ],
)

_soft_call = pl.kernel(
    _soft_body,
    out_type=jax.ShapeDtypeStruct((ROWS2P, CH), jnp.float32),
    mesh=_MESH,
    scratch_types=[
        pltpu.VMEM_SHARED((NPAD,), jnp.float32),
        pltpu.VMEM((RPS, CH), jnp.int32),
        pltpu.VMEM((RPS, CH), jnp.float32),
        pltpu.VMEM((RPW, CH), jnp.float32),
        pltpu.VMEM((RPT,), jnp.float32),
        pltpu.VMEM((CH,), jnp.float32),
        pltpu.VMEM((CH,), jnp.float32),
        pltpu.VMEM((L,), jnp.float32),
        pltpu.SemaphoreType.DMA,
        pltpu.SemaphoreType.DMA,
    ],
)

_scat_call = pl.kernel(
    _scat_body,
    out_type=jax.ShapeDtypeStruct((NC, NPAD, D), jnp.float32),
    mesh=_MESH,
    scratch_types=[
        pltpu.VMEM_SHARED((NPAD, D), jnp.float32),
        pltpu.VMEM((HALF, CH), jnp.int32),
        pltpu.VMEM((HALF, CH), jnp.float32),
        pltpu.VMEM((CH, D), jnp.float32),
        pltpu.VMEM((CH, D), jnp.float32),
        pltpu.VMEM((CH, D), jnp.float32),
        pltpu.SemaphoreType.DMA,
        pltpu.SemaphoreType.DMA,
    ],
)


@jax.jit
def _run(M, dest32, rev32, W0_w, W0_b, W1_w, W1_b, a_w, a_b):
  w1t = W1_w.T
  w0t = W0_w.T
  b01 = (W0_b + W1_b).reshape(1, D)
  aw = a_w.reshape(1, D)
  ab = a_b.reshape(1, 1)

  pad = ROWS2P * CH - E
  rev_p = jnp.concatenate(
      [rev32, jnp.zeros((pad,), jnp.int32)]).reshape(ROWS2P, CH)
  v = _gather_call(M, rev_p)

  s, gmax = _score_half(M, v, w0t, w1t, b01, aw, ab, BM, GRID_E, 0)

  s_p = jnp.concatenate(
      [s.reshape(E), jnp.full((pad,), -1e30, jnp.float32)]
  ).reshape(ROWS2P, CH)
  dest_p = jnp.concatenate(
      [dest32, jnp.zeros((pad,), jnp.int32)]).reshape(ROWS2P, CH)
  g16 = gmax.reshape(-1)[:L]
  alpha2 = _soft_call(s_p, dest_p, g16)
  partials = _scat_call(alpha2, dest_p, M)

  out = pl.pallas_call(
      _combine_body,
      grid=(5,),
      in_specs=[pl.BlockSpec((NC, 2000, D), lambda i: (0, i, 0))],
      out_specs=pl.BlockSpec((2000, D), lambda i: (i, 0)),
      out_shape=jax.ShapeDtypeStruct((N, D), jnp.float32),
  )(partials)

  return out, alpha2.reshape(-1)[:E].reshape(E, 1)


def kernel(M, dest, dim_size, rev_index, W0_w, W0_b, W1_w, W1_b, a_w, a_b):
  dest32 = dest.astype(jnp.int32)
  rev32 = rev_index.astype(jnp.int32)
  return _run(M, dest32, rev32, W0_w, W0_b, W1_w, W1_b, a_w, a_b)


# speedup vs baseline: 1.0915x; 1.0317x over previous
import functools

import jax
import jax.numpy as jnp
from jax import lax
from jax.experimental import pallas as pl
from jax.experimental.pallas import tpu as pltpu
from jax.experimental.pallas import tpu_sc as plsc

E = 320000
D = 128
N = 10000

NC = 2
NS = 16
NW = NC * NS
L = 16

CH = 80
ROWS2 = E // CH
ROWS2P = 4096
RPW = ROWS2P // NW
RPS = ROWS2P // NS
NPAD = 10240
RPT = NPAD // NS

HALF = 32

BM = 3200
GRID_E = E // BM


def _make_score_body(bm_rows):
  def _score_body(m_ref, v_ref, w0_ref, w1_ref, b_ref, aw_ref, ab_ref,
                  s_ref, g_ref):
    u = jnp.dot(m_ref[...], w0_ref[...], preferred_element_type=jnp.float32)
    u1 = jnp.dot(v_ref[...], w1_ref[...], preferred_element_type=jnp.float32)
    z = u + u1 + b_ref[...]
    z = jnp.where(z >= 0, z, 0.2 * z)
    s = jnp.sum(z * aw_ref[...], axis=1) + ab_ref[0, 0]
    s_ref[0] = s.reshape(bm_rows // 128, 128)
    bmx = jnp.full((8, 128), jnp.max(s))
    i = pl.program_id(0)

    @pl.when(i == 0)
    def _():
      g_ref[...] = bmx

    @pl.when(i != 0)
    def _():
      g_ref[...] = jnp.maximum(g_ref[...], bmx)

  return _score_body


def _score_half(m, v, w0t, w1t, b01, aw, ab, bm_rows, grid_n, blk_off):
  return pl.pallas_call(
      _make_score_body(bm_rows),
      grid=(grid_n,),
      in_specs=[
          pl.BlockSpec((bm_rows, D), lambda i: (i + blk_off, 0)),
          pl.BlockSpec((bm_rows, D), lambda i: (i, 0)),
          pl.BlockSpec((D, D), lambda i: (0, 0)),
          pl.BlockSpec((D, D), lambda i: (0, 0)),
          pl.BlockSpec((1, D), lambda i: (0, 0)),
          pl.BlockSpec((1, D), lambda i: (0, 0)),
          pl.BlockSpec(memory_space=pltpu.SMEM),
      ],
      out_specs=[
          pl.BlockSpec((1, bm_rows // 128, 128), lambda i: (i, 0, 0)),
          pl.BlockSpec((8, 128), lambda i: (0, 0)),
      ],
      out_shape=[
          jax.ShapeDtypeStruct((grid_n, bm_rows // 128, 128), jnp.float32),
          jax.ShapeDtypeStruct((8, 128), jnp.float32),
      ],
  )(m, v, w0t, w1t, b01, aw, ab)


def _combine_body(p_ref, o_ref):
  o_ref[...] = p_ref[0] + p_ref[1]


def _gather_body(p_hbm, rev_hbm, v_hbm, idx_v, rows4, gsem, wsem):
  c = lax.axis_index("c")
  s = lax.axis_index("s")
  w = s * NC + c
  base_row = w * RPW
  nv = jnp.minimum(RPW, ROWS2 - base_row)
  pltpu.sync_copy(rev_hbm.at[pl.ds(base_row, RPW)], idx_v)

  for b in range(3):
    pltpu.async_copy(p_hbm.at[idx_v.at[b]], rows4.at[b], gsem)

  @pl.loop(0, nv)
  def _(j):
    bj = lax.rem(j, 4)
    pltpu.make_async_copy(p_hbm.at[pl.ds(0, CH)], rows4.at[bj], gsem).wait()
    pltpu.async_copy(rows4.at[bj],
                     v_hbm.at[pl.ds((base_row + j) * CH, CH)], wsem)

    @pl.when(j >= 1)
    def _():
      pltpu.make_async_copy(
          rows4.at[bj], v_hbm.at[pl.ds(0, CH)], wsem).wait()

    @pl.when(j + 3 < nv)
    def _():
      b2 = lax.rem(j + 3, 4)
      pltpu.async_copy(p_hbm.at[idx_v.at[j + 3]], rows4.at[b2], gsem)

  pltpu.make_async_copy(rows4.at[0], v_hbm.at[pl.ds(0, CH)], wsem).wait()


def _soft_body(s_hbm, dest_hbm, g_hbm, alpha_hbm, seg_sm, dv, ev, av, zseg,
               den0, den1, gbuf, ssem, dsem):
  c = lax.axis_index("c")
  s = lax.axis_index("s")
  w = s * NC + c

  pltpu.sync_copy(g_hbm, gbuf)
  g16 = gbuf[...]
  zeros16 = jnp.zeros((L,), jnp.float32)
  for k in range(RPT // L):
    zseg[pl.ds(k * L, L)] = zeros16
  pltpu.sync_copy(zseg, seg_sm.at[pl.ds(s * RPT, RPT)])
  plsc.subcore_barrier()

  pltpu.sync_copy(s_hbm.at[pl.ds(s * RPS, RPS)], ev)
  pltpu.sync_copy(dest_hbm.at[pl.ds(s * RPS, RPS)], dv)

  @pl.loop(0, RPS)
  def _(k):
    for t in range(CH // L):
      ev[k, pl.ds(t * L, L)] = jnp.exp(ev[k, pl.ds(t * L, L)] - g16)
    pltpu.async_copy(ev.at[k], seg_sm.at[dv.at[k]], ssem, add=True)

    @pl.when(k >= 8)
    def _():
      pltpu.make_async_copy(ev.at[0], seg_sm.at[pl.ds(0, CH)], ssem).wait()

  for _ in range(8):
    pltpu.make_async_copy(ev.at[0], seg_sm.at[pl.ds(0, CH)], ssem).wait()

  plsc.subcore_barrier()

  base_row = w * RPW
  pltpu.sync_copy(s_hbm.at[pl.ds(base_row, RPW)], ev.at[pl.ds(0, RPW)])
  pltpu.sync_copy(dest_hbm.at[pl.ds(base_row, RPW)], dv.at[pl.ds(0, RPW)])
  dens = (den0, den1)
  pltpu.async_copy(seg_sm.at[dv.at[0]], den0, dsem)

  @pl.loop(0, RPW // 2)
  def _(jj):
    for k2 in range(2):
      j = 2 * jj + k2
      den = dens[k2]
      nxt = dens[1 - k2]
      pltpu.make_async_copy(seg_sm.at[pl.ds(0, CH)], den, dsem).wait()

      @pl.when(j + 1 < RPW)
      def _():
        pltpu.async_copy(seg_sm.at[dv.at[j + 1]], nxt, dsem)

      for t in range(CH // L):
        d16 = den[pl.ds(t * L, L)]
        e16 = jnp.exp(ev[j, pl.ds(t * L, L)] - g16)
        av[j, pl.ds(t * L, L)] = e16 / d16

  pltpu.sync_copy(av, alpha_hbm.at[pl.ds(base_row, RPW)])


def _scat_body(alpha_hbm, dest_hbm, m_hbm, part_hbm, out_sm, dv, av,
               mb0, mb1, mb2, lsem, ssem):
  c = lax.axis_index("c")
  s = lax.axis_index("s")
  w = s * NC + c
  mbufs = (mb0, mb1, mb2)

  zeros16 = jnp.zeros((L,), jnp.float32)

  @pl.loop(0, CH)
  def _(r):
    for q in range(D // L):
      mb0[r, pl.ds(q * L, L)] = zeros16

  for k in range(RPT // CH):
    pltpu.sync_copy(mb0, out_sm.at[pl.ds(s * RPT + k * CH, CH)])
  plsc.subcore_barrier()

  base_row = w * RPW
  nv = jnp.minimum(RPW, ROWS2 - base_row)

  for h in range(RPW // HALF):
    hbase = base_row + h * HALF
    nvh = jnp.clip(nv - h * HALF, 0, HALF)

    @pl.when(nvh > 0)
    def _():
      pltpu.sync_copy(alpha_hbm.at[pl.ds(hbase, HALF)], av)
      pltpu.sync_copy(dest_hbm.at[pl.ds(hbase, HALF)], dv)
      pltpu.async_copy(m_hbm.at[pl.ds(hbase * CH, CH)], mb0, lsem)

      @pl.loop(0, (HALF + 2) // 3)
      def _(jj):
        for k2 in range(3):
          j = 3 * jj + k2
          mb = mbufs[k2]

          @pl.when(j < HALF)
          def _():
            pltpu.make_async_copy(m_hbm.at[pl.ds(0, CH)], mb, lsem).wait()

            @pl.when(j >= 2)
            def _():
              pltpu.make_async_copy(
                  mb, out_sm.at[pl.ds(0, CH)], ssem).wait()

            @pl.when(j + 1 < HALF)
            def _():
              nxt = mbufs[(k2 + 1) % 3]
              pltpu.async_copy(m_hbm.at[pl.ds((hbase + j + 1) * CH, CH)],
                               nxt, lsem)

            def _scale_rows(g, carry):
              a16 = av[j, pl.ds(g * L, L)]
              for k in range(L):
                a_s = jnp.full((L,), a16[k], jnp.float32)
                r = g * L + k
                for q in range(D // L):
                  mb[r, pl.ds(q * L, L)] = mb[r, pl.ds(q * L, L)] * a_s
              return carry

            lax.fori_loop(0, CH // L, _scale_rows, 0)
            pltpu.async_copy(mb, out_sm.at[dv.at[j]], ssem, add=True)

      for _ in range(2):
        pltpu.make_async_copy(mb0, out_sm.at[pl.ds(0, CH)], ssem).wait()

  plsc.subcore_barrier()

  pltpu.sync_copy(out_sm.at[pl.ds(s * RPT, RPT)],
                  part_hbm.at[c].at[pl.ds(s * RPT, RPT)])


_MESH = plsc.VectorSubcoreMesh(core_axis_name="c", subcore_axis_name="s",
                               num_cores=NC, num_subcores=NS)

_gather_call = pl.kernel(
    _gather_body,
    out_type=jax.ShapeDtypeStruct((E, D), jnp.float32),
    mesh=_MESH,
    scratch_types=[
        pltpu.VMEM((RPW, CH), jnp.int32),
        pltpu.VMEM((4, CH, D), jnp.float32),
        pltpu.SemaphoreType.DMA,
        pltpu.SemaphoreType.DMA,
    ],
)

_soft_call = pl.kernel(
    _soft_body,
    out_type=jax.ShapeDtypeStruct((ROWS2P, CH), jnp.float32),
    mesh=_MESH,
    scratch_types=[
        pltpu.VMEM_SHARED((NPAD,), jnp.float32),
        pltpu.VMEM((RPS, CH), jnp.int32),
        pltpu.VMEM((RPS, CH), jnp.float32),
        pltpu.VMEM((RPW, CH), jnp.float32),
        pltpu.VMEM((RPT,), jnp.float32),
        pltpu.VMEM((CH,), jnp.float32),
        pltpu.VMEM((CH,), jnp.float32),
        pltpu.VMEM((L,), jnp.float32),
        pltpu.SemaphoreType.DMA,
        pltpu.SemaphoreType.DMA,
    ],
)

_scat_call = pl.kernel(
    _scat_body,
    out_type=jax.ShapeDtypeStruct((NC, NPAD, D), jnp.float32),
    mesh=_MESH,
    scratch_types=[
        pltpu.VMEM_SHARED((NPAD, D), jnp.float32),
        pltpu.VMEM((HALF, CH), jnp.int32),
        pltpu.VMEM((HALF, CH), jnp.float32),
        pltpu.VMEM((CH, D), jnp.float32),
        pltpu.VMEM((CH, D), jnp.float32),
        pltpu.VMEM((CH, D), jnp.float32),
        pltpu.SemaphoreType.DMA,
        pltpu.SemaphoreType.DMA,
    ],
)


@jax.jit
def _run(M, dest32, rev32, W0_w, W0_b, W1_w, W1_b, a_w, a_b):
  w1t = W1_w.T
  w0t = W0_w.T
  b01 = (W0_b + W1_b).reshape(1, D)
  aw = a_w.reshape(1, D)
  ab = a_b.reshape(1, 1)

  pad = ROWS2P * CH - E
  rev_p = jnp.concatenate(
      [rev32, jnp.zeros((pad,), jnp.int32)]).reshape(ROWS2P, CH)
  v = _gather_call(M, rev_p)

  s, gmax = _score_half(M, v, w0t, w1t, b01, aw, ab, BM, GRID_E, 0)

  s_p = jnp.concatenate(
      [s.reshape(E), jnp.full((pad,), -1e30, jnp.float32)]
  ).reshape(ROWS2P, CH)
  dest_p = jnp.concatenate(
      [dest32, jnp.zeros((pad,), jnp.int32)]).reshape(ROWS2P, CH)
  g16 = gmax.reshape(-1)[:L]
  alpha2 = _soft_call(s_p, dest_p, g16)
  partials = _scat_call(alpha2, dest_p, M)

  out = pl.pallas_call(
      _combine_body,
      grid=(5,),
      in_specs=[pl.BlockSpec((NC, 2000, D), lambda i: (0, i, 0))],
      out_specs=pl.BlockSpec((2000, D), lambda i: (i, 0)),
      out_shape=jax.ShapeDtypeStruct((N, D), jnp.float32),
  )(partials)

  return out, alpha2.reshape(-1)[:E].reshape(E, 1)


def kernel(M, dest, dim_size, rev_index, W0_w, W0_b, W1_w, W1_b, a_w, a_b):
  dest32 = dest.astype(jnp.int32)
  rev32 = rev_index.astype(jnp.int32)
  return _run(M, dest32, rev32, W0_w, W0_b, W1_w, W1_b, a_w, a_b)


# speedup vs baseline: 1.1746x; 1.0761x over previous
import functools

import jax
import jax.numpy as jnp
from jax import lax
from jax.experimental import pallas as pl
from jax.experimental.pallas import tpu as pltpu
from jax.experimental.pallas import tpu_sc as plsc

E = 320000
D = 128
N = 10000

NC = 2
NS = 16
NW = NC * NS
L = 16

CH = 80
ROWS2 = E // CH
ROWS2P = 4096
RPW = ROWS2P // NW
RPS = ROWS2P // NS
NPAD = 10240
RPT = NPAD // NS

HALF = 32

BM = 6400
GRID_E = E // BM


def _make_score_body(bm_rows):
  def _score_body(m_ref, v_ref, w0_ref, w1_ref, b_ref, aw_ref, ab_ref,
                  s_ref, g_ref):
    u = jnp.dot(m_ref[...], w0_ref[...], preferred_element_type=jnp.float32)
    u1 = jnp.dot(v_ref[...], w1_ref[...], preferred_element_type=jnp.float32)
    z = u + u1 + b_ref[...]
    z = jnp.where(z >= 0, z, 0.2 * z)
    s = jnp.sum(z * aw_ref[...], axis=1) + ab_ref[0, 0]
    s_ref[0] = s.reshape(bm_rows // 128, 128)
    bmx = jnp.full((8, 128), jnp.max(s))
    i = pl.program_id(0)

    @pl.when(i == 0)
    def _():
      g_ref[...] = bmx

    @pl.when(i != 0)
    def _():
      g_ref[...] = jnp.maximum(g_ref[...], bmx)

  return _score_body


def _score_half(m, v, w0t, w1t, b01, aw, ab, bm_rows, grid_n, blk_off):
  return pl.pallas_call(
      _make_score_body(bm_rows),
      grid=(grid_n,),
      in_specs=[
          pl.BlockSpec((bm_rows, D), lambda i: (i + blk_off, 0)),
          pl.BlockSpec((bm_rows, D), lambda i: (i, 0)),
          pl.BlockSpec((D, D), lambda i: (0, 0)),
          pl.BlockSpec((D, D), lambda i: (0, 0)),
          pl.BlockSpec((1, D), lambda i: (0, 0)),
          pl.BlockSpec((1, D), lambda i: (0, 0)),
          pl.BlockSpec(memory_space=pltpu.SMEM),
      ],
      out_specs=[
          pl.BlockSpec((1, bm_rows // 128, 128), lambda i: (i, 0, 0)),
          pl.BlockSpec((8, 128), lambda i: (0, 0)),
      ],
      out_shape=[
          jax.ShapeDtypeStruct((grid_n, bm_rows // 128, 128), jnp.float32),
          jax.ShapeDtypeStruct((8, 128), jnp.float32),
      ],
  )(m, v, w0t, w1t, b01, aw, ab)


def _combine_body(p_ref, o_ref):
  o_ref[...] = p_ref[0] + p_ref[1]


def _gather_body(p_hbm, rev_hbm, v_hbm, idx_v, rows4, gsem, wsem):
  c = lax.axis_index("c")
  s = lax.axis_index("s")
  w = s * NC + c
  base_row = w * RPW
  nv = jnp.minimum(RPW, ROWS2 - base_row)
  pltpu.sync_copy(rev_hbm.at[pl.ds(base_row, RPW)], idx_v)

  for b in range(3):
    pltpu.async_copy(p_hbm.at[idx_v.at[b]], rows4.at[b], gsem)

  @pl.loop(0, nv)
  def _(j):
    bj = lax.rem(j, 4)
    pltpu.make_async_copy(p_hbm.at[pl.ds(0, CH)], rows4.at[bj], gsem).wait()
    pltpu.async_copy(rows4.at[bj],
                     v_hbm.at[pl.ds((base_row + j) * CH, CH)], wsem)

    @pl.when(j >= 1)
    def _():
      pltpu.make_async_copy(
          rows4.at[bj], v_hbm.at[pl.ds(0, CH)], wsem).wait()

    @pl.when(j + 3 < nv)
    def _():
      b2 = lax.rem(j + 3, 4)
      pltpu.async_copy(p_hbm.at[idx_v.at[j + 3]], rows4.at[b2], gsem)

  pltpu.make_async_copy(rows4.at[0], v_hbm.at[pl.ds(0, CH)], wsem).wait()


def _soft_body(s_hbm, dest_hbm, g_hbm, alpha_hbm, seg_sm, dv, ev, av, zseg,
               den0, den1, gbuf, ssem, dsem):
  c = lax.axis_index("c")
  s = lax.axis_index("s")
  w = s * NC + c

  pltpu.sync_copy(g_hbm, gbuf)
  g16 = gbuf[...]
  zeros16 = jnp.zeros((L,), jnp.float32)
  for k in range(RPT // L):
    zseg[pl.ds(k * L, L)] = zeros16
  pltpu.sync_copy(zseg, seg_sm.at[pl.ds(s * RPT, RPT)])
  plsc.subcore_barrier()

  pltpu.sync_copy(s_hbm.at[pl.ds(s * RPS, RPS)], ev)
  pltpu.sync_copy(dest_hbm.at[pl.ds(s * RPS, RPS)], dv)

  @pl.loop(0, RPS)
  def _(k):
    for t in range(CH // L):
      ev[k, pl.ds(t * L, L)] = jnp.exp(ev[k, pl.ds(t * L, L)] - g16)
    pltpu.async_copy(ev.at[k], seg_sm.at[dv.at[k]], ssem, add=True)

    @pl.when(k >= 8)
    def _():
      pltpu.make_async_copy(ev.at[0], seg_sm.at[pl.ds(0, CH)], ssem).wait()

  for _ in range(8):
    pltpu.make_async_copy(ev.at[0], seg_sm.at[pl.ds(0, CH)], ssem).wait()

  plsc.subcore_barrier()

  base_row = w * RPW
  pltpu.sync_copy(s_hbm.at[pl.ds(base_row, RPW)], ev.at[pl.ds(0, RPW)])
  pltpu.sync_copy(dest_hbm.at[pl.ds(base_row, RPW)], dv.at[pl.ds(0, RPW)])
  dens = (den0, den1)
  pltpu.async_copy(seg_sm.at[dv.at[0]], den0, dsem)

  @pl.loop(0, RPW // 2)
  def _(jj):
    for k2 in range(2):
      j = 2 * jj + k2
      den = dens[k2]
      nxt = dens[1 - k2]
      pltpu.make_async_copy(seg_sm.at[pl.ds(0, CH)], den, dsem).wait()

      @pl.when(j + 1 < RPW)
      def _():
        pltpu.async_copy(seg_sm.at[dv.at[j + 1]], nxt, dsem)

      for t in range(CH // L):
        d16 = den[pl.ds(t * L, L)]
        e16 = jnp.exp(ev[j, pl.ds(t * L, L)] - g16)
        av[j, pl.ds(t * L, L)] = e16 / d16

  pltpu.sync_copy(av, alpha_hbm.at[pl.ds(base_row, RPW)])


def _scat_body(alpha_hbm, dest_hbm, m_hbm, part_hbm, out_sm, dv, av,
               mb0, mb1, mb2, lsem, ssem):
  c = lax.axis_index("c")
  s = lax.axis_index("s")
  w = s * NC + c
  mbufs = (mb0, mb1, mb2)

  zeros16 = jnp.zeros((L,), jnp.float32)

  @pl.loop(0, CH)
  def _(r):
    for q in range(D // L):
      mb0[r, pl.ds(q * L, L)] = zeros16

  for k in range(RPT // CH):
    pltpu.sync_copy(mb0, out_sm.at[pl.ds(s * RPT + k * CH, CH)])
  plsc.subcore_barrier()

  base_row = w * RPW
  nv = jnp.minimum(RPW, ROWS2 - base_row)

  for h in range(RPW // HALF):
    hbase = base_row + h * HALF
    nvh = jnp.clip(nv - h * HALF, 0, HALF)

    @pl.when(nvh > 0)
    def _():
      pltpu.sync_copy(alpha_hbm.at[pl.ds(hbase, HALF)], av)
      pltpu.sync_copy(dest_hbm.at[pl.ds(hbase, HALF)], dv)
      pltpu.async_copy(m_hbm.at[pl.ds(hbase * CH, CH)], mb0, lsem)

      @pl.loop(0, (HALF + 2) // 3)
      def _(jj):
        for k2 in range(3):
          j = 3 * jj + k2
          mb = mbufs[k2]

          @pl.when(j < HALF)
          def _():
            pltpu.make_async_copy(m_hbm.at[pl.ds(0, CH)], mb, lsem).wait()

            @pl.when(j >= 2)
            def _():
              pltpu.make_async_copy(
                  mb, out_sm.at[pl.ds(0, CH)], ssem).wait()

            @pl.when(j + 1 < HALF)
            def _():
              nxt = mbufs[(k2 + 1) % 3]
              pltpu.async_copy(m_hbm.at[pl.ds((hbase + j + 1) * CH, CH)],
                               nxt, lsem)

            def _scale_rows(g, carry):
              a16 = av[j, pl.ds(g * L, L)]
              for k in range(L):
                a_s = jnp.full((L,), a16[k], jnp.float32)
                r = g * L + k
                for q in range(D // L):
                  mb[r, pl.ds(q * L, L)] = mb[r, pl.ds(q * L, L)] * a_s
              return carry

            lax.fori_loop(0, CH // L, _scale_rows, 0)
            pltpu.async_copy(mb, out_sm.at[dv.at[j]], ssem, add=True)

      for _ in range(2):
        pltpu.make_async_copy(mb0, out_sm.at[pl.ds(0, CH)], ssem).wait()

  plsc.subcore_barrier()

  pltpu.sync_copy(out_sm.at[pl.ds(s * RPT, RPT)],
                  part_hbm.at[c].at[pl.ds(s * RPT, RPT)])


_MESH = plsc.VectorSubcoreMesh(core_axis_name="c", subcore_axis_name="s",
                               num_cores=NC, num_subcores=NS)

_gather_call = pl.kernel(
    _gather_body,
    out_type=jax.ShapeDtypeStruct((E, D), jnp.float32),
    mesh=_MESH,
    scratch_types=[
        pltpu.VMEM((RPW, CH), jnp.int32),
        pltpu.VMEM((4, CH, D), jnp.float32),
        pltpu.SemaphoreType.DMA,
        pltpu.SemaphoreType.DMA,
    ],
)

_soft_call = pl.kernel(
    _soft_body,
    out_type=jax.ShapeDtypeStruct((ROWS2P, CH), jnp.float32),
    mesh=_MESH,
    scratch_types=[
        pltpu.VMEM_SHARED((NPAD,), jnp.float32),
        pltpu.VMEM((RPS, CH), jnp.int32),
        pltpu.VMEM((RPS, CH), jnp.float32),
        pltpu.VMEM((RPW, CH), jnp.float32),
        pltpu.VMEM((RPT,), jnp.float32),
        pltpu.VMEM((CH,), jnp.float32),
        pltpu.VMEM((CH,), jnp.float32),
        pltpu.VMEM((L,), jnp.float32),
        pltpu.SemaphoreType.DMA,
        pltpu.SemaphoreType.DMA,
    ],
)

_scat_call = pl.kernel(
    _scat_body,
    out_type=jax.ShapeDtypeStruct((NC, NPAD, D), jnp.float32),
    mesh=_MESH,
    scratch_types=[
        pltpu.VMEM_SHARED((NPAD, D), jnp.float32),
        pltpu.VMEM((HALF, CH), jnp.int32),
        pltpu.VMEM((HALF, CH), jnp.float32),
        pltpu.VMEM((CH, D), jnp.float32),
        pltpu.VMEM((CH, D), jnp.float32),
        pltpu.VMEM((CH, D), jnp.float32),
        pltpu.SemaphoreType.DMA,
        pltpu.SemaphoreType.DMA,
    ],
)


@jax.jit
def _run(M, dest32, rev32, W0_w, W0_b, W1_w, W1_b, a_w, a_b):
  w1t = W1_w.T
  w0t = W0_w.T
  b01 = (W0_b + W1_b).reshape(1, D)
  aw = a_w.reshape(1, D)
  ab = a_b.reshape(1, 1)

  pad = ROWS2P * CH - E
  rev_p = jnp.concatenate(
      [rev32, jnp.zeros((pad,), jnp.int32)]).reshape(ROWS2P, CH)
  v = _gather_call(M, rev_p)

  s, gmax = _score_half(M, v, w0t, w1t, b01, aw, ab, BM, GRID_E, 0)

  s_p = jnp.concatenate(
      [s.reshape(E), jnp.full((pad,), -1e30, jnp.float32)]
  ).reshape(ROWS2P, CH)
  dest_p = jnp.concatenate(
      [dest32, jnp.zeros((pad,), jnp.int32)]).reshape(ROWS2P, CH)
  g16 = gmax.reshape(-1)[:L]
  alpha2 = _soft_call(s_p, dest_p, g16)
  partials = _scat_call(alpha2, dest_p, M)

  out = pl.pallas_call(
      _combine_body,
      grid=(5,),
      in_specs=[pl.BlockSpec((NC, 2000, D), lambda i: (0, i, 0))],
      out_specs=pl.BlockSpec((2000, D), lambda i: (i, 0)),
      out_shape=jax.ShapeDtypeStruct((N, D), jnp.float32),
  )(partials)

  return out, alpha2.reshape(-1)[:E].reshape(E, 1)


def kernel(M, dest, dim_size, rev_index, W0_w, W0_b, W1_w, W1_b, a_w, a_b):
  dest32 = dest.astype(jnp.int32)
  rev32 = rev_index.astype(jnp.int32)
  return _run(M, dest32, rev32, W0_w, W0_b, W1_w, W1_b, a_w, a_b)


# speedup vs baseline: 1.2050x; 1.0259x over previous
import functools

import jax
import jax.numpy as jnp
from jax import lax
from jax.experimental import pallas as pl
from jax.experimental.pallas import tpu as pltpu
from jax.experimental.pallas import tpu_sc as plsc

E = 320000
D = 128
N = 10000

NC = 2
NS = 16
NW = NC * NS
L = 16

CH = 80
ROWS2 = E // CH
ROWS2P = 4096
RPW = ROWS2P // NW
RPS = ROWS2P // NS
NPAD = 10240
RPT = NPAD // NS

HALF = 32

BM = 12800
GRID_E = E // BM


def _make_score_body(bm_rows):
  def _score_body(m_ref, v_ref, w0_ref, w1_ref, b_ref, aw_ref, ab_ref,
                  s_ref, g_ref):
    u = jnp.dot(m_ref[...], w0_ref[...], preferred_element_type=jnp.float32)
    u1 = jnp.dot(v_ref[...], w1_ref[...], preferred_element_type=jnp.float32)
    z = u + u1 + b_ref[...]
    z = jnp.where(z >= 0, z, 0.2 * z)
    s = jnp.sum(z * aw_ref[...], axis=1) + ab_ref[0, 0]
    s_ref[0] = s.reshape(bm_rows // 128, 128)
    bmx = jnp.full((8, 128), jnp.max(s))
    i = pl.program_id(0)

    @pl.when(i == 0)
    def _():
      g_ref[...] = bmx

    @pl.when(i != 0)
    def _():
      g_ref[...] = jnp.maximum(g_ref[...], bmx)

  return _score_body


def _score_half(m, v, w0t, w1t, b01, aw, ab, bm_rows, grid_n, blk_off):
  return pl.pallas_call(
      _make_score_body(bm_rows),
      grid=(grid_n,),
      in_specs=[
          pl.BlockSpec((bm_rows, D), lambda i: (i + blk_off, 0)),
          pl.BlockSpec((bm_rows, D), lambda i: (i, 0)),
          pl.BlockSpec((D, D), lambda i: (0, 0)),
          pl.BlockSpec((D, D), lambda i: (0, 0)),
          pl.BlockSpec((1, D), lambda i: (0, 0)),
          pl.BlockSpec((1, D), lambda i: (0, 0)),
          pl.BlockSpec(memory_space=pltpu.SMEM),
      ],
      out_specs=[
          pl.BlockSpec((1, bm_rows // 128, 128), lambda i: (i, 0, 0)),
          pl.BlockSpec((8, 128), lambda i: (0, 0)),
      ],
      out_shape=[
          jax.ShapeDtypeStruct((grid_n, bm_rows // 128, 128), jnp.float32),
          jax.ShapeDtypeStruct((8, 128), jnp.float32),
      ],
  )(m, v, w0t, w1t, b01, aw, ab)


def _combine_body(p_ref, o_ref):
  o_ref[...] = p_ref[0] + p_ref[1]


def _gather_body(p_hbm, rev_hbm, v_hbm, idx_v, rows4, gsem, wsem):
  c = lax.axis_index("c")
  s = lax.axis_index("s")
  w = s * NC + c
  base_row = w * RPW
  nv = jnp.minimum(RPW, ROWS2 - base_row)
  pltpu.sync_copy(rev_hbm.at[pl.ds(base_row, RPW)], idx_v)

  for b in range(3):
    pltpu.async_copy(p_hbm.at[idx_v.at[b]], rows4.at[b], gsem)

  @pl.loop(0, nv)
  def _(j):
    bj = lax.rem(j, 4)
    pltpu.make_async_copy(p_hbm.at[pl.ds(0, CH)], rows4.at[bj], gsem).wait()
    pltpu.async_copy(rows4.at[bj],
                     v_hbm.at[pl.ds((base_row + j) * CH, CH)], wsem)

    @pl.when(j >= 1)
    def _():
      pltpu.make_async_copy(
          rows4.at[bj], v_hbm.at[pl.ds(0, CH)], wsem).wait()

    @pl.when(j + 3 < nv)
    def _():
      b2 = lax.rem(j + 3, 4)
      pltpu.async_copy(p_hbm.at[idx_v.at[j + 3]], rows4.at[b2], gsem)

  pltpu.make_async_copy(rows4.at[0], v_hbm.at[pl.ds(0, CH)], wsem).wait()


def _soft_body(s_hbm, dest_hbm, g_hbm, alpha_hbm, seg_sm, dv, ev, av, zseg,
               den0, den1, gbuf, ssem, dsem):
  c = lax.axis_index("c")
  s = lax.axis_index("s")
  w = s * NC + c

  pltpu.sync_copy(g_hbm, gbuf)
  g16 = gbuf[...]
  zeros16 = jnp.zeros((L,), jnp.float32)
  for k in range(RPT // L):
    zseg[pl.ds(k * L, L)] = zeros16
  pltpu.sync_copy(zseg, seg_sm.at[pl.ds(s * RPT, RPT)])
  plsc.subcore_barrier()

  pltpu.sync_copy(s_hbm.at[pl.ds(s * RPS, RPS)], ev)
  pltpu.sync_copy(dest_hbm.at[pl.ds(s * RPS, RPS)], dv)

  @pl.loop(0, RPS)
  def _(k):
    for t in range(CH // L):
      ev[k, pl.ds(t * L, L)] = jnp.exp(ev[k, pl.ds(t * L, L)] - g16)
    pltpu.async_copy(ev.at[k], seg_sm.at[dv.at[k]], ssem, add=True)

    @pl.when(k >= 8)
    def _():
      pltpu.make_async_copy(ev.at[0], seg_sm.at[pl.ds(0, CH)], ssem).wait()

  for _ in range(8):
    pltpu.make_async_copy(ev.at[0], seg_sm.at[pl.ds(0, CH)], ssem).wait()

  plsc.subcore_barrier()

  base_row = w * RPW
  pltpu.sync_copy(s_hbm.at[pl.ds(base_row, RPW)], ev.at[pl.ds(0, RPW)])
  pltpu.sync_copy(dest_hbm.at[pl.ds(base_row, RPW)], dv.at[pl.ds(0, RPW)])
  dens = (den0, den1)
  pltpu.async_copy(seg_sm.at[dv.at[0]], den0, dsem)

  @pl.loop(0, RPW // 2)
  def _(jj):
    for k2 in range(2):
      j = 2 * jj + k2
      den = dens[k2]
      nxt = dens[1 - k2]
      pltpu.make_async_copy(seg_sm.at[pl.ds(0, CH)], den, dsem).wait()

      @pl.when(j + 1 < RPW)
      def _():
        pltpu.async_copy(seg_sm.at[dv.at[j + 1]], nxt, dsem)

      for t in range(CH // L):
        d16 = den[pl.ds(t * L, L)]
        e16 = jnp.exp(ev[j, pl.ds(t * L, L)] - g16)
        av[j, pl.ds(t * L, L)] = e16 / d16

  pltpu.sync_copy(av, alpha_hbm.at[pl.ds(base_row, RPW)])


def _scat_body(alpha_hbm, dest_hbm, m_hbm, part_hbm, out_sm, dv, av,
               mb0, mb1, mb2, lsem, ssem):
  c = lax.axis_index("c")
  s = lax.axis_index("s")
  w = s * NC + c
  mbufs = (mb0, mb1, mb2)

  zeros16 = jnp.zeros((L,), jnp.float32)

  @pl.loop(0, CH)
  def _(r):
    for q in range(D // L):
      mb0[r, pl.ds(q * L, L)] = zeros16

  for k in range(RPT // CH):
    pltpu.sync_copy(mb0, out_sm.at[pl.ds(s * RPT + k * CH, CH)])
  plsc.subcore_barrier()

  base_row = w * RPW
  nv = jnp.minimum(RPW, ROWS2 - base_row)

  for h in range(RPW // HALF):
    hbase = base_row + h * HALF
    nvh = jnp.clip(nv - h * HALF, 0, HALF)

    @pl.when(nvh > 0)
    def _():
      pltpu.sync_copy(alpha_hbm.at[pl.ds(hbase, HALF)], av)
      pltpu.sync_copy(dest_hbm.at[pl.ds(hbase, HALF)], dv)
      pltpu.async_copy(m_hbm.at[pl.ds(hbase * CH, CH)], mb0, lsem)

      @pl.loop(0, (HALF + 2) // 3)
      def _(jj):
        for k2 in range(3):
          j = 3 * jj + k2
          mb = mbufs[k2]

          @pl.when(j < HALF)
          def _():
            pltpu.make_async_copy(m_hbm.at[pl.ds(0, CH)], mb, lsem).wait()

            @pl.when(j >= 2)
            def _():
              pltpu.make_async_copy(
                  mb, out_sm.at[pl.ds(0, CH)], ssem).wait()

            @pl.when(j + 1 < HALF)
            def _():
              nxt = mbufs[(k2 + 1) % 3]
              pltpu.async_copy(m_hbm.at[pl.ds((hbase + j + 1) * CH, CH)],
                               nxt, lsem)

            def _scale_rows(g, carry):
              a16 = av[j, pl.ds(g * L, L)]
              for k in range(L):
                a_s = jnp.full((L,), a16[k], jnp.float32)
                r = g * L + k
                for q in range(D // L):
                  mb[r, pl.ds(q * L, L)] = mb[r, pl.ds(q * L, L)] * a_s
              return carry

            lax.fori_loop(0, CH // L, _scale_rows, 0)
            pltpu.async_copy(mb, out_sm.at[dv.at[j]], ssem, add=True)

      for _ in range(2):
        pltpu.make_async_copy(mb0, out_sm.at[pl.ds(0, CH)], ssem).wait()

  plsc.subcore_barrier()

  pltpu.sync_copy(out_sm.at[pl.ds(s * RPT, RPT)],
                  part_hbm.at[c].at[pl.ds(s * RPT, RPT)])


_MESH = plsc.VectorSubcoreMesh(core_axis_name="c", subcore_axis_name="s",
                               num_cores=NC, num_subcores=NS)

_gather_call = pl.kernel(
    _gather_body,
    out_type=jax.ShapeDtypeStruct((E, D), jnp.float32),
    mesh=_MESH,
    scratch_types=[
        pltpu.VMEM((RPW, CH), jnp.int32),
        pltpu.VMEM((4, CH, D), jnp.float32),
        pltpu.SemaphoreType.DMA,
        pltpu.SemaphoreType.DMA,
    ],
)

_soft_call = pl.kernel(
    _soft_body,
    out_type=jax.ShapeDtypeStruct((ROWS2P, CH), jnp.float32),
    mesh=_MESH,
    scratch_types=[
        pltpu.VMEM_SHARED((NPAD,), jnp.float32),
        pltpu.VMEM((RPS, CH), jnp.int32),
        pltpu.VMEM((RPS, CH), jnp.float32),
        pltpu.VMEM((RPW, CH), jnp.float32),
        pltpu.VMEM((RPT,), jnp.float32),
        pltpu.VMEM((CH,), jnp.float32),
        pltpu.VMEM((CH,), jnp.float32),
        pltpu.VMEM((L,), jnp.float32),
        pltpu.SemaphoreType.DMA,
        pltpu.SemaphoreType.DMA,
    ],
)

_scat_call = pl.kernel(
    _scat_body,
    out_type=jax.ShapeDtypeStruct((NC, NPAD, D), jnp.float32),
    mesh=_MESH,
    scratch_types=[
        pltpu.VMEM_SHARED((NPAD, D), jnp.float32),
        pltpu.VMEM((HALF, CH), jnp.int32),
        pltpu.VMEM((HALF, CH), jnp.float32),
        pltpu.VMEM((CH, D), jnp.float32),
        pltpu.VMEM((CH, D), jnp.float32),
        pltpu.VMEM((CH, D), jnp.float32),
        pltpu.SemaphoreType.DMA,
        pltpu.SemaphoreType.DMA,
    ],
)


@jax.jit
def _run(M, dest32, rev32, W0_w, W0_b, W1_w, W1_b, a_w, a_b):
  w1t = W1_w.T
  w0t = W0_w.T
  b01 = (W0_b + W1_b).reshape(1, D)
  aw = a_w.reshape(1, D)
  ab = a_b.reshape(1, 1)

  pad = ROWS2P * CH - E
  rev_p = jnp.concatenate(
      [rev32, jnp.zeros((pad,), jnp.int32)]).reshape(ROWS2P, CH)
  v = _gather_call(M, rev_p)

  s, gmax = _score_half(M, v, w0t, w1t, b01, aw, ab, BM, GRID_E, 0)

  s_p = jnp.concatenate(
      [s.reshape(E), jnp.full((pad,), -1e30, jnp.float32)]
  ).reshape(ROWS2P, CH)
  dest_p = jnp.concatenate(
      [dest32, jnp.zeros((pad,), jnp.int32)]).reshape(ROWS2P, CH)
  g16 = gmax.reshape(-1)[:L]
  alpha2 = _soft_call(s_p, dest_p, g16)
  partials = _scat_call(alpha2, dest_p, M)

  out = pl.pallas_call(
      _combine_body,
      grid=(5,),
      in_specs=[pl.BlockSpec((NC, 2000, D), lambda i: (0, i, 0))],
      out_specs=pl.BlockSpec((2000, D), lambda i: (i, 0)),
      out_shape=jax.ShapeDtypeStruct((N, D), jnp.float32),
  )(partials)

  return out, alpha2.reshape(-1)[:E].reshape(E, 1)


def kernel(M, dest, dim_size, rev_index, W0_w, W0_b, W1_w, W1_b, a_w, a_b):
  dest32 = dest.astype(jnp.int32)
  rev32 = rev_index.astype(jnp.int32)
  return _run(M, dest32, rev32, W0_w, W0_b, W1_w, W1_b, a_w, a_b)
